# baseline (speedup 1.0000x reference)
import functools

import jax
import jax.numpy as jnp
from jax.experimental import pallas as pl
from jax.experimental.pallas import tpu as pltpu


def _embed_kernel(x_ref, w_ref, padd_ref, out_ref):
    x = x_ref[0].astype(jnp.bfloat16)
    w = w_ref[...].astype(jnp.bfloat16)
    acc = jnp.dot(x, w, preferred_element_type=jnp.float32)
    out_ref[0] = acc + padd_ref[0]


@functools.partial(jax.jit, static_argnames=("tb", "tn", "interpret"))
def _embed(seq_t, W_tok, padd, tb=2048, tn=1024, interpret=False):
    L, B, C = seq_t.shape
    D = W_tok.shape[1]
    grid = (L, B // tb, D // tn)
    out = pl.pallas_call(
        _embed_kernel,
        grid=grid,
        in_specs=[
            pl.BlockSpec((1, tb, C), lambda l, i, j: (l, i, 0)),
            pl.BlockSpec((C, tn), lambda l, i, j: (0, j)),
            pl.BlockSpec((1, 1, tn), lambda l, i, j: (l, 0, j)),
        ],
        out_specs=pl.BlockSpec((1, tb, tn), lambda l, i, j: (l, i, j)),
        out_shape=jax.ShapeDtypeStruct((L, B, D), jnp.float32),
        compiler_params=pltpu.CompilerParams(
            dimension_semantics=("parallel", "parallel", "parallel"),
        ),
        interpret=interpret,
    )(seq_t, W_tok, padd)
    return out


def kernel(sequence, W_tok, b_tok, pos_table):
    B, L, C = sequence.shape
    D = W_tok.shape[1]
    padd = (pos_table + b_tok[None, :]).reshape(L, 1, D)
    seq_t = jnp.transpose(sequence, (1, 0, 2))
    out_t = _embed(seq_t, W_tok, padd)
    x = jnp.transpose(out_t, (1, 0, 2))
    mask = jnp.ones((B, L), dtype=bool)
    return (x, mask)


# speedup vs baseline: 1.2254x; 1.2254x over previous
import functools

import jax
import jax.numpy as jnp
from jax.experimental import pallas as pl
from jax.experimental.pallas import tpu as pltpu


def _embed_kernel(x_ref, w_ref, padd_ref, out_ref):
    x = x_ref[0].astype(jnp.bfloat16)
    w = w_ref[...].astype(jnp.bfloat16)
    acc = jnp.dot(x, w, preferred_element_type=jnp.float32)
    out_ref[0] = acc + padd_ref[0]


@functools.partial(jax.jit, static_argnames=("tb", "tn", "interpret"))
def _embed(seq_t, W_tok, padd, tb=2048, tn=2048, interpret=False):
    L, B, C = seq_t.shape
    D = W_tok.shape[1]
    grid = (L, B // tb, D // tn)
    out = pl.pallas_call(
        _embed_kernel,
        grid=grid,
        in_specs=[
            pl.BlockSpec((1, tb, C), lambda l, i, j: (l, i, 0)),
            pl.BlockSpec((C, tn), lambda l, i, j: (0, j)),
            pl.BlockSpec((1, 1, tn), lambda l, i, j: (l, 0, j)),
        ],
        out_specs=pl.BlockSpec((1, tb, tn), lambda l, i, j: (l, i, j)),
        out_shape=jax.ShapeDtypeStruct((L, B, D), jnp.float32),
        compiler_params=pltpu.CompilerParams(
            dimension_semantics=("parallel", "parallel", "parallel"),
        ),
        interpret=interpret,
    )(seq_t, W_tok, padd)
    return out


def kernel(sequence, W_tok, b_tok, pos_table):
    B, L, C = sequence.shape
    D = W_tok.shape[1]
    padd = (pos_table + b_tok[None, :]).reshape(L, 1, D)
    seq_t = jnp.transpose(sequence, (1, 0, 2))
    out_t = _embed(seq_t, W_tok, padd)
    x = jnp.transpose(out_t, (1, 0, 2))
    mask = jnp.ones((B, L), dtype=bool)
    return (x, mask)
